# baseline (speedup 1.0000x reference)
import jax
import jax.numpy as jnp
from jax import lax
from jax.experimental import pallas as pl
from jax.experimental.pallas import tpu as pltpu
from jax.experimental.pallas import tpu_sc as plsc

NC = 2
NS = 16
NW = NC * NS
LANES = 16
EMBED = 64
CHUNK = 128
SCALE = 8.0


def _make_sc_kernel(n_chunks):
    mesh = plsc.VectorSubcoreMesh(core_axis_name="c", subcore_axis_name="s")

    def body(idx_hbm, table_hbm, out_hbm, idx_v, rows_v, sem):
        wid = lax.axis_index("s") * NC + lax.axis_index("c")
        pltpu.sync_copy(idx_hbm.at[wid], idx_v)

        def chunk_body(j, carry):
            pltpu.async_copy(table_hbm.at[idx_v.at[j]], rows_v, sem).wait()

            def scale_row(i, c2):
                for t in range(EMBED // LANES):
                    sl = pl.ds(t * LANES, LANES)
                    rows_v[i, sl] = rows_v[i, sl] * SCALE
                return c2

            lax.fori_loop(0, CHUNK, scale_row, 0)
            pltpu.sync_copy(rows_v, out_hbm.at[wid, j])
            return carry

        lax.fori_loop(0, n_chunks, chunk_body, 0)

    return pl.kernel(
        body,
        out_type=jax.ShapeDtypeStruct((NW, n_chunks, CHUNK, EMBED), jnp.float32),
        mesh=mesh,
        scratch_types=[
            pltpu.VMEM((n_chunks, CHUNK), jnp.int32),
            pltpu.VMEM((CHUNK, EMBED), jnp.float32),
            pltpu.SemaphoreType.DMA,
        ],
        compiler_params=pltpu.CompilerParams(use_tc_tiling_on_sc=False),
    )


def kernel(input, table):
    b, t = input.shape
    n_total = b * t
    n_chunks = n_total // (NW * CHUNK)
    idx = input.reshape(NW, n_chunks, CHUNK).astype(jnp.int32)
    out = _make_sc_kernel(n_chunks)(idx, table)
    return out.reshape(b, t, EMBED)


# speedup vs baseline: 1.2078x; 1.2078x over previous
import jax
import jax.numpy as jnp
from jax import lax
from jax.experimental import pallas as pl
from jax.experimental.pallas import tpu as pltpu
from jax.experimental.pallas import tpu_sc as plsc

NC = 2
NS = 16
NW = NC * NS
LANES = 16
EMBED = 64
CHUNK = 128
NBUF = 4
SCALE = 8.0


def _make_sc_kernel(n_chunks):
    mesh = plsc.VectorSubcoreMesh(core_axis_name="c", subcore_axis_name="s")
    n_groups = n_chunks // NBUF

    def body(idx_hbm, table_hbm, out_hbm, idx_v, in_v, out_v, *sems):
        gsems = sems[:NBUF]
        ssems = sems[NBUF:]
        wid = lax.axis_index("s") * NC + lax.axis_index("c")
        pltpu.sync_copy(idx_hbm.at[wid], idx_v)

        def gather(j, b):
            pltpu.async_copy(table_hbm.at[idx_v.at[j]], in_v.at[b], gsems[b])

        def gather_wait(j, b):
            pltpu.make_async_copy(
                table_hbm.at[idx_v.at[j]], in_v.at[b], gsems[b]).wait()

        def scatter(j, b):
            pltpu.async_copy(out_v.at[b], out_hbm.at[wid, j], ssems[b])

        def scatter_wait(j, b):
            pltpu.make_async_copy(
                out_v.at[b], out_hbm.at[wid, j], ssems[b]).wait()

        for b in range(NBUF):
            gather(b, b)

        def group_body(g, carry):
            for b in range(NBUF):
                j = g * NBUF + b
                gather_wait(j, b)

                @pl.when(g >= 1)
                def _():
                    scatter_wait(j - NBUF, b)

                @plsc.parallel_loop(0, CHUNK, 1, unroll=4)
                def _(i):
                    for t in range(EMBED // LANES):
                        sl = pl.ds(t * LANES, LANES)
                        out_v[b, i, sl] = in_v[b, i, sl] * SCALE

                @pl.when(g < n_groups - 1)
                def _():
                    gather(j + NBUF, b)

                scatter(j, b)
            return carry

        lax.fori_loop(0, n_groups, group_body, 0)

        for b in range(NBUF):
            scatter_wait(n_chunks - NBUF + b, b)

    return pl.kernel(
        body,
        out_type=jax.ShapeDtypeStruct((NW, n_chunks, CHUNK, EMBED), jnp.float32),
        mesh=mesh,
        scratch_types=[
            pltpu.VMEM((n_chunks, CHUNK), jnp.int32),
            pltpu.VMEM((NBUF, CHUNK, EMBED), jnp.float32),
            pltpu.VMEM((NBUF, CHUNK, EMBED), jnp.float32),
        ] + [pltpu.SemaphoreType.DMA] * (2 * NBUF),
        compiler_params=pltpu.CompilerParams(use_tc_tiling_on_sc=False),
    )


def kernel(input, table):
    b, t = input.shape
    n_total = b * t
    n_chunks = n_total // (NW * CHUNK)
    idx = input.reshape(NW, n_chunks, CHUNK).astype(jnp.int32)
    out = _make_sc_kernel(n_chunks)(idx, table)
    return out.reshape(b, t, EMBED)
